# baseline (speedup 1.0000x reference)
import functools

import jax
import jax.numpy as jnp
from jax import lax
from jax.experimental import pallas as pl
from jax.experimental.pallas import tpu as pltpu
from jax.experimental.pallas import tpu_sc as plsc

_B, _L = 4096, 50
_DW, _DF = 128, 64
_DO = _DW + _DF
_NW = 32
_BPW = _B // _NW
_GB = 2
_GT = _GB * _L
_NG = _BPW // _GB

_mesh = plsc.VectorSubcoreMesh(core_axis_name="c", subcore_axis_name="s")


def _make_stack_kernel(nb):
    bpw = nb // _NW
    ng = bpw // _GB

    @functools.partial(
        pl.kernel,
        out_type=jax.ShapeDtypeStruct((nb, _L, _DO), jnp.float32),
        mesh=_mesh,
        scratch_types=[
            pltpu.VMEM((ng, _GT), jnp.int32),
            pltpu.VMEM((ng, _GT), jnp.int32),
            [pltpu.VMEM((_GT, _DW), jnp.float32)] * 2,
            [pltpu.VMEM((_GT, _DW), jnp.float32)] * 2,
            [pltpu.VMEM((_GB, _L, _DO), jnp.float32)] * 2,
            [pltpu.SemaphoreType.DMA] * 2,
            [pltpu.SemaphoreType.DMA] * 2,
            [pltpu.SemaphoreType.DMA] * 2,
        ],
    )
    def stack_kernel(word_hbm, feat_hbm, ww_hbm, wf_hbm, out_hbm,
                     idxw_v, idxf_v, rw, rf, asm, semw, semf, semo):
        wid = lax.axis_index("s") * 2 + lax.axis_index("c")
        pltpu.sync_copy(word_hbm.at[wid], idxw_v)
        pltpu.sync_copy(feat_hbm.at[wid], idxf_v)

        def fire(c, b):
            pltpu.async_copy(ww_hbm.at[idxw_v.at[c]], rw[b], semw[b])
            pltpu.async_copy(wf_hbm.at[idxf_v.at[c]], rf[b], semf[b])

        def process(c, b, first):
            pltpu.make_async_copy(ww_hbm.at[idxw_v.at[c]], rw[b], semw[b]).wait()
            pltpu.make_async_copy(wf_hbm.at[idxf_v.at[c]], rf[b], semf[b]).wait()
            @pl.when(jnp.logical_not(first))
            def _():
                pltpu.make_async_copy(
                    asm[b], out_hbm.at[pl.ds(0, _GB)], semo[b]).wait()

            for g in range(_GB):
                @pl.loop(0, _L)
                def _row(l):
                    r = g * _L + l
                    for k in range(_DW // 16):
                        asm[b][g, l, pl.ds(16 * k, 16)] = rw[b][r, pl.ds(16 * k, 16)]
                    for k in range(_DF // 16):
                        asm[b][g, l, pl.ds(_DW + 16 * k, 16)] = rf[b][r, pl.ds(16 * k, 16)]

            b0 = wid * bpw + c * _GB
            pltpu.async_copy(asm[b], out_hbm.at[pl.ds(b0, _GB)], semo[b])

        fire(0, 0)

        @pl.loop(0, ng // 2)
        def _pair(p):
            c0 = 2 * p
            fire(c0 + 1, 1)
            process(c0, 0, p == 0)

            @pl.when(p < ng // 2 - 1)
            def _():
                fire(c0 + 2, 0)
            process(c0 + 1, 1, p == 0)

        pltpu.make_async_copy(asm[0], out_hbm.at[pl.ds(0, _GB)], semo[0]).wait()
        pltpu.make_async_copy(asm[1], out_hbm.at[pl.ds(0, _GB)], semo[1]).wait()

    return stack_kernel


_NSPLIT = 2
_NBS = _B // _NSPLIT
_slab_kernel = _make_stack_kernel(_NBS)


def kernel(word, feat, W_word, W_feat):
    wf_pad = jnp.pad(W_feat, ((0, 0), (0, _DW - _DF)))
    word_i = word.astype(jnp.int32)
    feat_i = feat.astype(jnp.int32)
    halves = []
    for s in range(_NSPLIT):
        w3 = lax.slice_in_dim(word_i, s * _NBS, (s + 1) * _NBS, axis=0)
        f3 = lax.slice_in_dim(feat_i, s * _NBS, (s + 1) * _NBS, axis=0)
        w3 = w3.reshape(_NW, (_NBS // _NW) // _GB, _GT)
        f3 = f3.reshape(_NW, (_NBS // _NW) // _GB, _GT)
        halves.append(_slab_kernel(w3, f3, W_word, wf_pad))
    return jnp.concatenate(halves, axis=0)


# speedup vs baseline: 1.4416x; 1.4416x over previous
import functools

import jax
import jax.numpy as jnp
from jax import lax
from jax.experimental import pallas as pl
from jax.experimental.pallas import tpu as pltpu
from jax.experimental.pallas import tpu_sc as plsc

_B, _L = 4096, 50
_DW, _DF = 128, 64
_DO = _DW + _DF
_NW = 32
_BPW = _B // _NW
_NG = _BPW
_ND = 4

_mesh = plsc.VectorSubcoreMesh(core_axis_name="c", subcore_axis_name="s")


@functools.partial(
    pl.kernel,
    out_type=jax.ShapeDtypeStruct((_B, _L, _DO), jnp.float32),
    mesh=_mesh,
    scratch_types=[
        [pltpu.VMEM((2, _L), jnp.int32)] * _ND,
        [pltpu.VMEM((_L, _DW), jnp.float32)] * _ND,
        [pltpu.VMEM((1, _L, _DO), jnp.float32)] * _ND,
        [pltpu.SemaphoreType.DMA] * _ND,
        [pltpu.SemaphoreType.DMA] * _ND,
        [pltpu.SemaphoreType.DMA] * _ND,
        [pltpu.SemaphoreType.DMA] * _ND,
    ],
)
def _stack_kernel(idx_hbm, ww_hbm, wf_hbm, out_hbm,
                  cix, rf, asm, semi, semw, semf, semo):
    wid = lax.axis_index("s") * 2 + lax.axis_index("c")

    def stage_idx(c, s):
        pltpu.async_copy(idx_hbm.at[wid * _NG + c], cix[s], semi[s])

    def wait_idx(s):
        pltpu.make_async_copy(idx_hbm.at[0], cix[s], semi[s]).wait()

    def fire(c, b):
        pltpu.async_copy(ww_hbm.at[cix[b].at[0]],
                         asm[b].at[0, :, pl.ds(0, _DW)], semw[b])
        pltpu.async_copy(wf_hbm.at[cix[b].at[1]], rf[b], semf[b])

    def wait_write(b):
        pltpu.make_async_copy(asm[b], out_hbm.at[pl.ds(0, 1)], semo[b]).wait()

    def process(c, b):
        bn, bs = (b + 1) % _ND, (b + 2) % _ND
        @pl.when(c >= 3)
        def _():
            wait_write(bn)

        @pl.when(c + 2 < _NG)
        def _():
            stage_idx(c + 2, bs)

        @pl.when(c + 1 < _NG)
        def _():
            wait_idx(bn)
            fire(c + 1, bn)

        pltpu.make_async_copy(ww_hbm.at[cix[b].at[0]],
                              asm[b].at[0, :, pl.ds(0, _DW)], semw[b]).wait()
        pltpu.make_async_copy(wf_hbm.at[cix[b].at[1]], rf[b], semf[b]).wait()

        @pl.loop(0, _L, unroll=5)
        def _row(l):
            for k in range(_DF // 16):
                asm[b][0, l, pl.ds(_DW + 16 * k, 16)] = rf[b][l, pl.ds(16 * k, 16)]

        pltpu.async_copy(asm[b], out_hbm.at[pl.ds(wid * _NG + c, 1)], semo[b])

    stage_idx(0, 0)
    stage_idx(1, 1)
    wait_idx(0)
    fire(0, 0)

    @pl.loop(0, _NG // _ND)
    def _quad(p):
        c0 = _ND * p
        for i in range(_ND):
            process(c0 + i, i)

    for c in (_NG - 3, _NG - 2, _NG - 1):
        wait_write(c % _ND)


def kernel(word, feat, W_word, W_feat):
    wg = word.reshape(_B, 1, _L).astype(jnp.int32)
    fg = feat.reshape(_B, 1, _L).astype(jnp.int32)
    idx = jnp.concatenate([wg, fg], axis=1)
    wf_pad = jnp.pad(W_feat, ((0, 0), (0, _DW - _DF)))
    return _stack_kernel(idx, W_word, wf_pad)
